# baseline (speedup 1.0000x reference)
import functools

import jax
import jax.numpy as jnp
from jax import lax
from jax.experimental import pallas as pl
from jax.experimental.pallas import tpu as pltpu
from jax.experimental.pallas import tpu_sc as plsc

_V = 100000
_D = 300
_B = 1024
_L = 50

_LANES = 16
_DPAD = 304
_NSL = _DPAD // _LANES
_LP = 64
_LG = 56


def _sc_pool(table_pad, idx_pad):
    info = plsc.get_sparse_core_info()
    nw = info.num_cores * info.num_subcores
    bpw = _B // nw

    mesh = plsc.VectorSubcoreMesh(core_axis_name="c", subcore_axis_name="s")

    @functools.partial(
        pl.kernel,
        mesh=mesh,
        compiler_params=pltpu.CompilerParams(
            needs_layout_passes=False, use_tc_tiling_on_sc=False),
        out_type=jax.ShapeDtypeStruct((_B, _DPAD), jnp.float32),
        scratch_types=[
            pltpu.VMEM((bpw, _LP), jnp.int32),
            pltpu.VMEM((_LG, _DPAD), jnp.float32),
            pltpu.VMEM((bpw, _DPAD), jnp.float32),
            pltpu.SemaphoreType.DMA,
        ],
    )
    def k(table_hbm, idx_hbm, out_hbm, idx_v, rows_v, xbuf, sem):
        wid = lax.axis_index("s") * info.num_cores + lax.axis_index("c")
        base = wid * bpw
        pltpu.sync_copy(idx_hbm.at[pl.ds(base, bpw)], idx_v)
        lanes = lax.iota(jnp.int32, _LANES)

        def item(i, carry):
            pltpu.async_copy(
                table_hbm.at[idx_v.at[i, pl.ds(0, _LG)]], rows_v, sem).wait()

            def row(j, accs):
                sq = jnp.zeros((_LANES,), jnp.float32)
                for s in range(_NSL):
                    v = rows_v[j, pl.ds(s * _LANES, _LANES)]
                    sq = sq + v * v
                sv = sq
                for sh in (8, 4, 2, 1):
                    perm = lanes ^ sh
                    sv = sv + jnp.take_along_axis(sv, perm, axis=0)
                bits = plsc.bitcast(sv, jnp.int32)
                y = plsc.bitcast(jnp.int32(0x5F3759DF) - (bits >> 1), jnp.float32)
                for _ in range(3):
                    y = y * (1.5 - 0.5 * sv * y * y)
                nrm = sv * y
                scale = jnp.where(nrm > 1.0, 1.0 / (nrm + 1e-7), 1.0)
                new = []
                for s in range(_NSL):
                    v = rows_v[j, pl.ds(s * _LANES, _LANES)]
                    new.append(accs[s] + v * scale)
                return tuple(new)

            accs = lax.fori_loop(
                0, _L, row,
                tuple(jnp.zeros((_LANES,), jnp.float32) for _ in range(_NSL)),
            )
            inv = jnp.float32(1.0 / _L)
            for s in range(_NSL):
                xbuf[i, pl.ds(s * _LANES, _LANES)] = accs[s] * inv
            return carry

        lax.fori_loop(0, bpw, item, 0)
        pltpu.sync_copy(xbuf, out_hbm.at[pl.ds(base, bpw)])

    return k(table_pad, idx_pad)


_VT = 2048


def _project(x_pad, w, b):
    def mm(x_ref, w_ref, b_ref, o_ref):
        o_ref[...] = lax.dot_general(
            x_ref[:, :_D], w_ref[...], (((1,), (1,)), ((), ())),
            preferred_element_type=jnp.float32,
        ) + b_ref[...]

    return pl.pallas_call(
        mm,
        grid=(pl.cdiv(_V, _VT),),
        in_specs=[
            pl.BlockSpec((_B, _DPAD), lambda v: (0, 0)),
            pl.BlockSpec((_VT, _D), lambda v: (v, 0)),
            pl.BlockSpec((1, _VT), lambda v: (0, v)),
        ],
        out_specs=pl.BlockSpec((_B, _VT), lambda v: (0, v)),
        out_shape=jax.ShapeDtypeStruct((_B, _V), jnp.float32),
    )(x_pad, w, b.reshape(1, _V))


def kernel(inputs_, emb_table, lin_w, lin_b):
    table_pad = jnp.pad(emb_table, ((0, 0), (0, _DPAD - _D)))
    idx_pad = jnp.pad(inputs_, ((0, 0), (0, _LP - _L)))
    x_pad = _sc_pool(table_pad, idx_pad)
    return _project(x_pad, lin_w, lin_b)


# speedup vs baseline: 1.1165x; 1.1165x over previous
import functools

import jax
import jax.numpy as jnp
from jax import lax
from jax.experimental import pallas as pl
from jax.experimental.pallas import tpu as pltpu
from jax.experimental.pallas import tpu_sc as plsc

_V = 100000
_D = 300
_B = 1024
_L = 50

_LANES = 16
_DPAD = 384
_GPR = _DPAD // 128
_NSL = _DPAD // _LANES
_W = 152


def _sc_pool(table3, idx3):
    info = plsc.get_sparse_core_info()
    nw = info.num_cores * info.num_subcores
    bpw = _B // nw

    mesh = plsc.VectorSubcoreMesh(core_axis_name="c", subcore_axis_name="s")

    @functools.partial(
        pl.kernel,
        mesh=mesh,
        compiler_params=pltpu.CompilerParams(
            needs_layout_passes=False, use_tc_tiling_on_sc=False),
        out_type=jax.ShapeDtypeStruct((_B, _DPAD), jnp.float32),
        scratch_types=[
            pltpu.VMEM((bpw, _W), jnp.int32),
            pltpu.VMEM((_W, 128), jnp.float32),
            pltpu.VMEM((_W, 128), jnp.float32),
            pltpu.VMEM((bpw, _DPAD), jnp.float32),
            pltpu.SemaphoreType.DMA,
            pltpu.SemaphoreType.DMA,
        ],
    )
    def k(table_hbm, idx_hbm, out_hbm, idx_v, buf_a, buf_b, xbuf, sem_a,
          sem_b):
        wid = lax.axis_index("s") * info.num_cores + lax.axis_index("c")
        base = wid * bpw
        pltpu.sync_copy(idx_hbm.at[pl.ds(base, bpw)], idx_v)
        lanes = lax.iota(jnp.int32, _LANES)
        bufs = (buf_a, buf_b)
        sems = (sem_a, sem_b)

        def start(item, b):
            pltpu.async_copy(
                table_hbm.at[idx_v.at[item, pl.ds(0, _W)]], bufs[b], sems[b])

        def drain(b):
            pltpu.make_async_copy(
                table_hbm.at[pl.ds(0, _W)], bufs[b], sems[b]).wait()

        def process(item, buf):
            def row(j, accs):
                g = _GPR * j
                sq = jnp.zeros((_LANES,), jnp.float32)
                for t in range(_GPR):
                    for h in range(8):
                        v = buf[g + t, pl.ds(h * _LANES, _LANES)]
                        sq = sq + v * v
                sv = sq
                for sh in (8, 4, 2, 1):
                    perm = lanes ^ sh
                    sv = sv + jnp.take_along_axis(sv, perm, axis=0)
                bits = plsc.bitcast(sv, jnp.int32)
                y = plsc.bitcast(jnp.int32(0x5F3759DF) - (bits >> 1),
                                 jnp.float32)
                for _ in range(3):
                    y = y * (1.5 - 0.5 * sv * y * y)
                nrm = sv * y
                scale = jnp.where(nrm > 1.0, 1.0 / (nrm + 1e-7), 1.0)
                new = []
                for t in range(_GPR):
                    for h in range(8):
                        v = buf[g + t, pl.ds(h * _LANES, _LANES)]
                        new.append(accs[8 * t + h] + v * scale)
                return tuple(new)

            accs = lax.fori_loop(
                0, _L, row,
                tuple(jnp.zeros((_LANES,), jnp.float32) for _ in range(_NSL)),
            )
            inv = jnp.float32(1.0 / _L)
            for s in range(_NSL):
                xbuf[item, pl.ds(s * _LANES, _LANES)] = accs[s] * inv

        start(0, 0)
        start(1, 1)

        def body(i2, carry):
            for b in range(2):
                item = 2 * i2 + b
                drain(b)
                process(item, bufs[b])
                nxt = jnp.minimum(item + 2, bpw - 1)
                start(nxt, b)
            return carry

        lax.fori_loop(0, bpw // 2, body, 0)
        drain(0)
        drain(1)
        pltpu.sync_copy(xbuf, out_hbm.at[pl.ds(base, bpw)])

    return k(table3, idx3)


_VT = 2048


def _project(x_pad, w, b2):
    def mm(x_ref, w_ref, b_ref, o_ref):
        o_ref[...] = lax.dot_general(
            x_ref[:, :_D], w_ref[...], (((1,), (1,)), ((), ())),
            preferred_element_type=jnp.float32,
        ) + b_ref[...]

    return pl.pallas_call(
        mm,
        grid=(pl.cdiv(_V, _VT),),
        in_specs=[
            pl.BlockSpec((_B, _DPAD), lambda v: (0, 0)),
            pl.BlockSpec((_VT, _D), lambda v: (v, 0)),
            pl.BlockSpec((1, _VT), lambda v: (0, v)),
        ],
        out_specs=pl.BlockSpec((_B, _VT), lambda v: (0, v)),
        out_shape=jax.ShapeDtypeStruct((_B, _V), jnp.float32),
    )(x_pad, w, b2)


def kernel(inputs_, emb_table, lin_w, lin_b):
    table3 = jnp.pad(emb_table, ((0, 0), (0, _DPAD - _D)))
    table3 = table3.reshape(_V * _GPR, 128)
    idx = inputs_.astype(jnp.int32)
    idx3 = (idx[:, :, None] * _GPR +
            jnp.arange(_GPR, dtype=jnp.int32)[None, None, :])
    idx3 = idx3.reshape(_B, _L * _GPR)
    idx3 = jnp.pad(idx3, ((0, 0), (0, _W - _L * _GPR)))
    x_pad = _sc_pool(table3, idx3)
    return _project(x_pad, lin_w, lin_b.reshape(1, _V))
